# Initial kernel scaffold; baseline (speedup 1.0000x reference)
import jax
import jax.numpy as jnp
from jax.experimental import pallas as pl


def kernel(x, edge_index, batch, W1l, b1, W1r, W2l, b2, W2r):
    raise NotImplementedError("write your pallas kernel here")


# speedup vs baseline: 7.3387x; 7.3387x over previous
import jax
import jax.numpy as jnp
from jax import lax
from jax.experimental import pallas as pl
from jax.experimental.pallas import tpu as pltpu
from jax.experimental.pallas import tpu_sc as plsc

N = 10000
E = 320000
DIN = 128
H = 64
G = 8
NC = 2
NS = 16
GRP = 128
GPT = 80
EPC = GPT * NS * GRP
E_HALF = E // 2
PAD = EPC - E_HALF
STRIPE = 632
ACC_ROWS = NS * STRIPE


def _make_agg(with_counts):
    mesh = plsc.VectorSubcoreMesh(core_axis_name="c", subcore_axis_name="s")
    out_type = [jax.ShapeDtypeStruct((NC, ACC_ROWS, H), jnp.float32)]
    scratch = [
        pltpu.VMEM((GPT, GRP), jnp.int32),
        pltpu.VMEM((GPT, GRP), jnp.int32),
        pltpu.VMEM((GRP, H), jnp.float32),
        pltpu.VMEM((GRP, H), jnp.float32),
        pltpu.VMEM_SHARED((ACC_ROWS, H), jnp.float32),
        pltpu.SemaphoreType.DMA,
        pltpu.SemaphoreType.DMA,
    ]
    if with_counts:
        out_type.append(jax.ShapeDtypeStruct((ACC_ROWS,), jnp.float32))
        out_type.append(jax.ShapeDtypeStruct((ACC_ROWS,), jnp.float32))
        scratch += [
            pltpu.VMEM((GRP,), jnp.float32),
            pltpu.VMEM_SHARED((ACC_ROWS,), jnp.float32),
            pltpu.VMEM((STRIPE,), jnp.float32),
        ]

    def body(y_hbm, src_hbm, dst_hbm, zsum_hbm, *rest):
        if with_counts:
            (out_s, out_c0, out_c1, src_v, dst_v, msg0, msg1, acc,
             sem0, sem1, ones_v, cnt_acc, cbuf) = rest
        else:
            (out_s, src_v, dst_v, msg0, msg1, acc, sem0, sem1) = rest
        c = lax.axis_index("c")
        s = lax.axis_index("s")

        pltpu.sync_copy(zsum_hbm.at[pl.ds(s * STRIPE, STRIPE)],
                        acc.at[pl.ds(s * STRIPE, STRIPE)])
        if with_counts:
            for k in range(STRIPE // 16):
                cbuf[pl.ds(16 * k, 16)] = jnp.zeros((16,), jnp.float32)
            cbuf[pl.ds(STRIPE - 16, 16)] = jnp.zeros((16,), jnp.float32)
            pltpu.sync_copy(cbuf, cnt_acc.at[pl.ds(s * STRIPE, STRIPE)])
            for k in range(GRP // 16):
                ones_v[pl.ds(16 * k, 16)] = jnp.full((16,), 1.0, jnp.float32)

        row0 = c * (NS * GPT) + s * GPT
        pltpu.sync_copy(src_hbm.at[pl.ds(row0, GPT)], src_v)
        pltpu.sync_copy(dst_hbm.at[pl.ds(row0, GPT)], dst_v)
        plsc.subcore_barrier()

        def fire(g, buf, sem):
            pltpu.async_copy(y_hbm.at[src_v.at[g]], buf, sem)

        def wait(buf, sem):
            pltpu.make_async_copy(y_hbm.at[src_v.at[0]], buf, sem).wait()

        def scat(g, buf):
            pltpu.sync_copy(buf, acc.at[dst_v.at[g]], add=True)
            if with_counts:
                pltpu.sync_copy(ones_v, cnt_acc.at[dst_v.at[g]], add=True)

        fire(0, msg0, sem0)

        def loop_body(i, carry):
            g0 = 2 * i
            fire(g0 + 1, msg1, sem1)
            wait(msg0, sem0)
            scat(g0, msg0)
            fire(g0 + 2, msg0, sem0)
            wait(msg1, sem1)
            scat(g0 + 1, msg1)
            return carry

        lax.fori_loop(0, GPT // 2 - 1, loop_body, 0)
        fire(GPT - 1, msg1, sem1)
        wait(msg0, sem0)
        scat(GPT - 2, msg0)
        wait(msg1, sem1)
        scat(GPT - 1, msg1)

        plsc.subcore_barrier()
        pltpu.sync_copy(acc.at[pl.ds(s * STRIPE, STRIPE)],
                        out_s.at[c, pl.ds(s * STRIPE, STRIPE)])
        if with_counts:
            pltpu.sync_copy(cnt_acc.at[pl.ds(s * STRIPE, STRIPE)], cbuf)

            @pl.when(c == 0)
            def _():
                pltpu.sync_copy(cbuf, out_c0.at[pl.ds(s * STRIPE, STRIPE)])

            @pl.when(c == 1)
            def _():
                pltpu.sync_copy(cbuf, out_c1.at[pl.ds(s * STRIPE, STRIPE)])

    return pl.kernel(
        body, out_type=out_type, mesh=mesh, scratch_types=scratch,
        compiler_params=pltpu.CompilerParams(use_tc_tiling_on_sc=False))


def _mm_pre(x_ref, wl_ref, wr_ref, y_ref, r_ref):
    xb = x_ref[...]
    y_ref[...] = jnp.dot(xb, wl_ref[...], preferred_element_type=jnp.float32)
    r_ref[...] = jnp.dot(xb, wr_ref[...], preferred_element_type=jnp.float32)


def _mm_mid(ps_ref, c0_ref, c1_ref, r_ref, b_ref, wl_ref, wr_ref,
            y_ref, r2_ref):
    ps = ps_ref[0] + ps_ref[1]
    cn = c0_ref[0, 0] + c1_ref[0, 0]
    agg = ps * (1.0 / jnp.maximum(cn, 1.0))[:, None]
    h = jnp.maximum(agg + b_ref[...] + r_ref[...], 0.0)
    y_ref[...] = jnp.dot(h, wl_ref[...], preferred_element_type=jnp.float32)
    r2_ref[...] = jnp.dot(h, wr_ref[...], preferred_element_type=jnp.float32)


def _mm_fin(ps_ref, c0_ref, c1_ref, r_ref, b_ref, batch_ref, out_ref, acc_ref):
    i = pl.program_id(0)
    ps = ps_ref[0] + ps_ref[1]
    cn = c0_ref[0, 0] + c1_ref[0, 0]
    agg = ps * (1.0 / jnp.maximum(cn, 1.0))[:, None]
    h = jnp.maximum(agg + b_ref[...] + r_ref[...], 0.0)
    he = jnp.concatenate([h, jnp.ones((1000, H), jnp.float32)], axis=1)
    b = batch_ref[0, 0]
    gids = lax.broadcasted_iota(jnp.int32, (G, 1000), 0)
    mask = (b[None, :] == gids).astype(jnp.float32)
    contrib = jnp.dot(mask, he, preferred_element_type=jnp.float32)

    @pl.when(i == 0)
    def _():
        acc_ref[...] = contrib

    @pl.when(i > 0)
    def _():
        acc_ref[...] = acc_ref[...] + contrib

    @pl.when(i == 9)
    def _():
        out_ref[...] = acc_ref[:, :H] / jnp.maximum(acc_ref[:, H:], 1.0)


@jax.jit
def kernel(x, edge_index, batch, W1l, b1, W1r, W2l, b2, W2r):
    f32 = jnp.float32
    src = edge_index[0].astype(jnp.int32)
    dst = edge_index[1].astype(jnp.int32)
    pad0 = jnp.zeros((PAD,), jnp.int32)
    padN = jnp.full((PAD,), N, jnp.int32)
    src_p = jnp.concatenate([src[:E_HALF], pad0, src[E_HALF:], pad0]
                            ).reshape(NC * NS * GPT, GRP)
    dst_p = jnp.concatenate([dst[:E_HALF], padN, dst[E_HALF:], padN]
                            ).reshape(NC * NS * GPT, GRP)
    zsum = jnp.zeros((ACC_ROWS, H), f32)

    BR = 1000
    nb = N // BR

    y1, r1 = pl.pallas_call(
        _mm_pre,
        grid=(nb,),
        in_specs=[
            pl.BlockSpec((BR, DIN), lambda i: (i, 0)),
            pl.BlockSpec((DIN, H), lambda i: (0, 0)),
            pl.BlockSpec((DIN, H), lambda i: (0, 0)),
        ],
        out_specs=[
            pl.BlockSpec((BR, H), lambda i: (i, 0)),
            pl.BlockSpec((BR, H), lambda i: (i, 0)),
        ],
        out_shape=[jax.ShapeDtypeStruct((N, H), f32)] * 2,
    )(x, W1l, W1r)

    agg1 = _make_agg(with_counts=True)
    psum1, cnt0, cnt1 = agg1(y1, src_p, dst_p, zsum)
    cnt0_r = cnt0[:N].reshape(nb, 1, BR)
    cnt1_r = cnt1[:N].reshape(nb, 1, BR)

    y2, r2 = pl.pallas_call(
        _mm_mid,
        grid=(nb,),
        in_specs=[
            pl.BlockSpec((NC, BR, H), lambda i: (0, i, 0)),
            pl.BlockSpec((1, 1, BR), lambda i: (i, 0, 0)),
            pl.BlockSpec((1, 1, BR), lambda i: (i, 0, 0)),
            pl.BlockSpec((BR, H), lambda i: (i, 0)),
            pl.BlockSpec((1, H), lambda i: (0, 0)),
            pl.BlockSpec((H, H), lambda i: (0, 0)),
            pl.BlockSpec((H, H), lambda i: (0, 0)),
        ],
        out_specs=[
            pl.BlockSpec((BR, H), lambda i: (i, 0)),
            pl.BlockSpec((BR, H), lambda i: (i, 0)),
        ],
        out_shape=[jax.ShapeDtypeStruct((N, H), f32)] * 2,
    )(psum1, cnt0_r, cnt1_r, r1, b1.reshape(1, H), W2l, W2r)

    agg2 = _make_agg(with_counts=False)
    res2 = agg2(y2, src_p, dst_p, zsum)
    psum2 = res2[0] if isinstance(res2, (list, tuple)) else res2

    batch_r = batch.astype(jnp.int32).reshape(nb, 1, BR)
    pooled = pl.pallas_call(
        _mm_fin,
        grid=(nb,),
        in_specs=[
            pl.BlockSpec((NC, BR, H), lambda i: (0, i, 0)),
            pl.BlockSpec((1, 1, BR), lambda i: (i, 0, 0)),
            pl.BlockSpec((1, 1, BR), lambda i: (i, 0, 0)),
            pl.BlockSpec((BR, H), lambda i: (i, 0)),
            pl.BlockSpec((1, H), lambda i: (0, 0)),
            pl.BlockSpec((1, 1, BR), lambda i: (i, 0, 0)),
        ],
        out_specs=pl.BlockSpec((G, H), lambda i: (0, 0)),
        out_shape=jax.ShapeDtypeStruct((G, H), f32),
        scratch_shapes=[pltpu.VMEM((G, 2 * H), f32)],
    )(psum2, cnt0_r, cnt1_r, r2, b2.reshape(1, H), batch_r)

    return pooled
